# baseline (speedup 1.0000x reference)
import functools

import jax
import jax.numpy as jnp
from jax import lax
from jax.experimental import pallas as pl
from jax.experimental.pallas import tpu as pltpu
from jax.experimental.pallas import tpu_sc as plsc

B = 16384
D = 64
N = 1000000
NC = 2
NS = 16
NW = NC * NS
BPW = B // NW
KCH = BPW // 128

_mesh = plsc.VectorSubcoreMesh(core_axis_name="c", subcore_axis_name="s")


def _transpose_body(in_ref, o_ref):
    o_ref[...] = in_ref[...].T


def _tc_transpose(tT):
    blk = 2048
    grid = (N + blk - 1) // blk
    return pl.pallas_call(
        _transpose_body,
        grid=(grid,),
        in_specs=[pl.BlockSpec((D, blk), lambda i: (0, i))],
        out_specs=pl.BlockSpec((blk, D), lambda i: (i, 0)),
        out_shape=jax.ShapeDtypeStruct((N, D), jnp.float32),
    )(tT)


@functools.partial(
    pl.kernel,
    mesh=_mesh,
    compiler_params=pltpu.CompilerParams(use_tc_tiling_on_sc=False),
    out_type=[
        jax.ShapeDtypeStruct((B, D), jnp.float32),
        jax.ShapeDtypeStruct((B, D), jnp.float32),
    ],
    scratch_types=[
        pltpu.VMEM((KCH, 128), jnp.int32),
        pltpu.VMEM((KCH, 128), jnp.int32),
        pltpu.VMEM((BPW, D), jnp.float32),
        pltpu.VMEM((BPW, D), jnp.float32),
        pltpu.SemaphoreType.DMA,
    ],
)
def _sc_gather(uid_hbm, iid_hbm, ut_hbm, it_hbm, uout_hbm, iout_hbm,
               uidx_v, iidx_v, urows_v, irows_v, sem):
    wid = lax.axis_index("s") * NC + lax.axis_index("c")
    pltpu.sync_copy(uid_hbm.at[wid], uidx_v)
    pltpu.sync_copy(iid_hbm.at[wid], iidx_v)
    copies = []
    for k in range(KCH):
        copies.append(pltpu.async_copy(
            ut_hbm.at[uidx_v.at[k]], urows_v.at[pl.ds(k * 128, 128)], sem))
        copies.append(pltpu.async_copy(
            it_hbm.at[iidx_v.at[k]], irows_v.at[pl.ds(k * 128, 128)], sem))
    for c in copies:
        c.wait()
    base = wid * BPW
    pltpu.sync_copy(urows_v, uout_hbm.at[pl.ds(base, BPW)])
    pltpu.sync_copy(irows_v, iout_hbm.at[pl.ds(base, BPW)])


def _head_body(u_ref, v_ref, w_ref, b_ref, o_ref):
    m = u_ref[...] * v_ref[...]
    p = jnp.sum(m * w_ref[...], axis=1, keepdims=True) + b_ref[0]
    o_ref[...] = 1.0 / (1.0 + jnp.exp(-p))


def _head(u, v, w_row, b):
    blk = 2048
    return pl.pallas_call(
        _head_body,
        grid=(B // blk,),
        in_specs=[
            pl.BlockSpec((blk, D), lambda i: (i, 0)),
            pl.BlockSpec((blk, D), lambda i: (i, 0)),
            pl.BlockSpec((1, D), lambda i: (0, 0)),
            pl.BlockSpec(memory_space=pltpu.SMEM),
        ],
        out_specs=pl.BlockSpec((blk, 1), lambda i: (i, 0)),
        out_shape=jax.ShapeDtypeStruct((B, 1), jnp.float32),
    )(u, v, w_row, b)


def kernel(user_id, item_id, user_table, item_table, W, b):
    uid = user_id.reshape(NW, KCH, 128).astype(jnp.int32)
    iid = item_id.reshape(NW, KCH, 128).astype(jnp.int32)
    u_rm = _tc_transpose(user_table.T)
    u_rows, i_rows = _sc_gather(uid, iid, u_rm, item_table)
    return _head(u_rows, i_rows, W.reshape(1, D), b)


# speedup vs baseline: 2.9478x; 2.9478x over previous
import functools

import jax
import jax.numpy as jnp
from jax import lax
from jax.experimental import pallas as pl
from jax.experimental.pallas import tpu as pltpu
from jax.experimental.pallas import tpu_sc as plsc

B = 16384
D = 64
N = 1000000
NC = 2
NS = 16
NW = NC * NS
BPW = B // NW
NBUF = 8

_mesh = plsc.VectorSubcoreMesh(core_axis_name="c", subcore_axis_name="s")


@functools.partial(
    pl.kernel,
    mesh=_mesh,
    compiler_params=pltpu.CompilerParams(needs_layout_passes=False),
    out_type=[
        jax.ShapeDtypeStruct((B * D,), jnp.float32),
        jax.ShapeDtypeStruct((B * D,), jnp.float32),
    ],
    scratch_types=[
        pltpu.VMEM((BPW * 16,), jnp.int32),
        pltpu.VMEM((BPW * 16,), jnp.int32),
        pltpu.VMEM((BPW * 16,), jnp.int32),
        pltpu.VMEM((BPW * 16,), jnp.int32),
        pltpu.VMEM((BPW * D,), jnp.float32),
    ] + [pltpu.VMEM((D, 128), jnp.float32) for _ in range(NBUF)]
      + [pltpu.SemaphoreType.DMA for _ in range(NBUF)],
)
def _sc_slab_gather(ucs_hbm, uls_hbm, ics_hbm, ils_hbm, utT_hbm, itT_hbm,
                    uo_hbm, io_hbm,
                    ucs_v, uls_v, ics_v, ils_v, rows_v, *slots_and_sems):
    slots = slots_and_sems[:NBUF]
    sems = slots_and_sems[NBUF:]
    wid = lax.axis_index("s") * NC + lax.axis_index("c")
    pltpu.sync_copy(ucs_hbm.at[wid], ucs_v)
    pltpu.sync_copy(uls_hbm.at[wid], uls_v)
    pltpu.sync_copy(ics_hbm.at[wid], ics_v)
    pltpu.sync_copy(ils_hbm.at[wid], ils_v)
    lane16 = lax.iota(jnp.int32, 16)

    def do_table(tT_hbm, out_hbm, cs_v, ls_v, slots):

        def issue(k, b):
            s = cs_v[pl.ds(pl.multiple_of(k * 16, 16), 16)][0]
            pltpu.async_copy(
                tT_hbm.at[:, pl.ds(pl.multiple_of(s, 128), 128)],
                slots[b], sems[b])

        def extract(k, b):
            pltpu.make_async_copy(
                tT_hbm.at[:, pl.ds(0, 128)], slots[b], sems[b]).wait()
            lvec = ls_v[pl.ds(pl.multiple_of(k * 16, 16), 16)]
            for j in range(4):
                v = plsc.load_gather(slots[b], [lane16 + 16 * j, lvec])
                rows_v[pl.ds(pl.multiple_of(k * D + 16 * j, 16), 16)] = v

        for b in range(NBUF):
            issue(b, b)

        def body(k0):
            for half in range(2):
                for b4 in range(4):
                    b = half * 4 + b4
                    extract(k0 + b, b)
                for b4 in range(4):
                    b = half * 4 + b4

                    @pl.when(k0 + NBUF + b < BPW)
                    def _():
                        issue(k0 + NBUF + b, b)

        pl.loop(0, BPW, step=NBUF)(body)
        pltpu.sync_copy(rows_v, out_hbm.at[pl.ds(wid * BPW * D, BPW * D)])

    do_table(utT_hbm, uo_hbm, ucs_v, uls_v, slots)
    do_table(itT_hbm, io_hbm, ics_v, ils_v, slots)


def _head_body(u_ref, v_ref, w_ref, b_ref, o_ref):
    m = u_ref[...] * v_ref[...]
    p = jnp.sum(m * w_ref[...], axis=1, keepdims=True) + b_ref[0]
    o_ref[...] = 1.0 / (1.0 + jnp.exp(-p))


def _head(u, v, w_row, b):
    blk = 2048
    return pl.pallas_call(
        _head_body,
        grid=(B // blk,),
        in_specs=[
            pl.BlockSpec((blk, D), lambda i: (i, 0)),
            pl.BlockSpec((blk, D), lambda i: (i, 0)),
            pl.BlockSpec((1, D), lambda i: (0, 0)),
            pl.BlockSpec(memory_space=pltpu.SMEM),
        ],
        out_specs=pl.BlockSpec((blk, 1), lambda i: (i, 0)),
        out_shape=jax.ShapeDtypeStruct((B, 1), jnp.float32),
    )(u, v, w_row, b)


def kernel(user_id, item_id, user_table, item_table, W, b):
    uid = user_id.astype(jnp.int32)
    iid = item_id.astype(jnp.int32)
    def rep16(x):
        return jnp.broadcast_to(x.reshape(NW, BPW, 1),
                                (NW, BPW, 16)).reshape(NW, BPW * 16)
    ucs = rep16(uid & ~127)
    uls = rep16(uid & 127)
    ics = rep16(iid & ~127)
    ils = rep16(iid & 127)
    u1d, i1d = _sc_slab_gather(ucs, uls, ics, ils,
                               user_table.T, item_table.T)
    return _head(u1d.reshape(B, D), i1d.reshape(B, D), W.reshape(1, D), b)


# speedup vs baseline: 3.2301x; 1.0958x over previous
import functools

import jax
import jax.numpy as jnp
from jax import lax
from jax.experimental import pallas as pl
from jax.experimental.pallas import tpu as pltpu
from jax.experimental.pallas import tpu_sc as plsc

B = 16384
D = 64
N = 1000000
NC = 2
NS = 16
NW = NC * NS
BPW = B // NW
NBUF = 8

_mesh = plsc.VectorSubcoreMesh(core_axis_name="c", subcore_axis_name="s")


@functools.partial(
    pl.kernel,
    mesh=_mesh,
    compiler_params=pltpu.CompilerParams(needs_layout_passes=False),
    out_type=[
        jax.ShapeDtypeStruct((B * D,), jnp.float32),
        jax.ShapeDtypeStruct((B * D,), jnp.float32),
    ],
    scratch_types=[
        pltpu.VMEM((BPW * 16,), jnp.int32),
        pltpu.VMEM((BPW * 16,), jnp.int32),
        pltpu.VMEM((BPW * 16,), jnp.int32),
        pltpu.VMEM((BPW * 16,), jnp.int32),
        pltpu.VMEM((BPW * D,), jnp.float32),
    ] + [pltpu.VMEM((D, 128), jnp.float32) for _ in range(NBUF)]
      + [pltpu.SemaphoreType.DMA for _ in range(NBUF)],
)
def _sc_slab_gather(ucs_hbm, uls_hbm, ics_hbm, ils_hbm, utT_hbm, itT_hbm,
                    uo_hbm, io_hbm,
                    ucs_v, uls_v, ics_v, ils_v, rows_v, *slots_and_sems):
    slots = slots_and_sems[:NBUF]
    sems = slots_and_sems[NBUF:]
    wid = lax.axis_index("s") * NC + lax.axis_index("c")
    pltpu.sync_copy(ucs_hbm.at[wid], ucs_v)
    pltpu.sync_copy(uls_hbm.at[wid], uls_v)
    pltpu.sync_copy(ics_hbm.at[wid], ics_v)
    pltpu.sync_copy(ils_hbm.at[wid], ils_v)
    lane16 = lax.iota(jnp.int32, 16)

    def do_table(tT_hbm, out_hbm, cs_v, ls_v, slots):

        def issue(k, b):
            s = cs_v[pl.ds(pl.multiple_of(k * 16, 16), 16)][0]
            pltpu.async_copy(
                tT_hbm.at[:, pl.ds(pl.multiple_of(s, 128), 128)],
                slots[b], sems[b])

        def extract(k, b):
            pltpu.make_async_copy(
                tT_hbm.at[:, pl.ds(0, 128)], slots[b], sems[b]).wait()
            lvec = ls_v[pl.ds(pl.multiple_of(k * 16, 16), 16)]
            for j in range(4):
                v = plsc.load_gather(slots[b], [lane16 + 16 * j, lvec])
                rows_v[pl.ds(pl.multiple_of(k * D + 16 * j, 16), 16)] = v

        for b in range(NBUF):
            issue(b, b)

        def body(k0):
            for b in range(NBUF):
                extract(k0 + b, b)

                @pl.when(k0 + NBUF + b < BPW)
                def _():
                    issue(k0 + NBUF + b, b)

        pl.loop(0, BPW, step=NBUF)(body)
        pltpu.sync_copy(rows_v, out_hbm.at[pl.ds(wid * BPW * D, BPW * D)])

    do_table(utT_hbm, uo_hbm, ucs_v, uls_v, slots)
    do_table(itT_hbm, io_hbm, ics_v, ils_v, slots)


def _head_body(u_ref, v_ref, w_ref, b_ref, o_ref):
    m = u_ref[...] * v_ref[...]
    p = jnp.sum(m * w_ref[...], axis=1, keepdims=True) + b_ref[0]
    o_ref[...] = 1.0 / (1.0 + jnp.exp(-p))


def _head(u, v, w_row, b):
    blk = 2048
    return pl.pallas_call(
        _head_body,
        grid=(B // blk,),
        in_specs=[
            pl.BlockSpec((blk, D), lambda i: (i, 0)),
            pl.BlockSpec((blk, D), lambda i: (i, 0)),
            pl.BlockSpec((1, D), lambda i: (0, 0)),
            pl.BlockSpec(memory_space=pltpu.SMEM),
        ],
        out_specs=pl.BlockSpec((blk, 1), lambda i: (i, 0)),
        out_shape=jax.ShapeDtypeStruct((B, 1), jnp.float32),
    )(u, v, w_row, b)


def kernel(user_id, item_id, user_table, item_table, W, b):
    uid = user_id.astype(jnp.int32)
    iid = item_id.astype(jnp.int32)
    def rep16(x):
        return jnp.broadcast_to(x.reshape(NW, BPW, 1),
                                (NW, BPW, 16)).reshape(NW, BPW * 16)
    ucs = rep16(uid & ~127)
    uls = rep16(uid & 127)
    ics = rep16(iid & ~127)
    ils = rep16(iid & 127)
    u1d, i1d = _sc_slab_gather(ucs, uls, ics, ils,
                               user_table.T, item_table.T)
    return _head(u1d.reshape(B, D), i1d.reshape(B, D), W.reshape(1, D), b)
